# baseline (speedup 1.0000x reference)
import functools

import jax
import jax.numpy as jnp
from jax import lax
from jax.experimental import pallas as pl
from jax.experimental.pallas import tpu as pltpu
from jax.experimental.pallas import tpu_sc as plsc

_H = 16
_S = 2048
_NB = 15
_RB = _S // 8
_W = 4088
_VAR0 = 1984
_FILL_R = 2096

_mesh = plsc.VectorSubcoreMesh(
    core_axis_name="c", subcore_axis_name="s", num_cores=2, num_subcores=16
)


@functools.partial(
    pl.kernel,
    out_type=jax.ShapeDtypeStruct((_H, _S, _S), jnp.float32),
    mesh=_mesh,
    scratch_types=[
        pltpu.VMEM((_NB * _NB * _H,), jnp.float32),
        pltpu.VMEM((8, _W), jnp.float32),
        pltpu.VMEM_SHARED((16, 8, _W), jnp.float32),
        pltpu.SemaphoreType.DMA,
    ],
    compiler_params=pltpu.CompilerParams(
        needs_layout_passes=False, use_tc_tiling_on_sc=False
    ),
)
def _rpb_sc(table_hbm, out_hbm, table_v, strip_v, strips_sh, sem):
    h = lax.axis_index("s")
    half = lax.axis_index("c")

    pltpu.sync_copy(table_hbm, table_v)

    iota = lax.iota(jnp.int32, 16)
    fj = iota & 7
    ksub = iota >> 3

    def tab_idx(fi, fjv, idx1):
        return (fi - fjv + 7) * (_NB * _H) + idx1 * _H + h

    pat_l = [plsc.load_gather(table_v, [tab_idx(fi, fj, 14)]) for fi in range(8)]
    pat_r = [plsc.load_gather(table_v, [tab_idx(fi, fj, 0)]) for fi in range(8)]

    for fi in range(8):
        for u in range(7):
            idx1 = 14 - (2 * u + ksub)
            vals = plsc.load_gather(table_v, [tab_idx(fi, fj, idx1)])
            strip_v[fi, pl.ds(_VAR0 + 16 * u, 16)] = vals

    def fill_body(t, carry):
        c = 16 * t
        for fi in range(8):
            strip_v[fi, pl.ds(c, 16)] = pat_l[fi]
            strip_v[fi, pl.ds(_FILL_R + c, 16)] = pat_r[fi]
        return carry

    lax.fori_loop(0, _VAR0 // 16, fill_body, 0)
    for fi in range(8):
        strip_v[fi, pl.ds(4072, 16)] = pat_r[fi]

    pltpu.sync_copy(strip_v, strips_sh.at[h])

    r0 = half * (_RB // 2)

    def window_copy(ri):
        c0 = 8 * (_RB - 1 - ri)
        return pltpu.make_async_copy(
            strips_sh.at[h, :, pl.ds(c0, _S)],
            out_hbm.at[h, pl.ds(8 * ri, 8), :],
            sem,
        )

    def write_body(t, carry):
        window_copy(r0 + t).start()

        @pl.when(t > 0)
        def _():
            window_copy(r0).wait()

        return carry

    lax.fori_loop(0, _RB // 2, write_body, 0)
    window_copy(r0).wait()


def kernel(seq_len, table):
    del seq_len
    out = _rpb_sc(table.reshape(-1))
    return out[None]


# speedup vs baseline: 1.1448x; 1.1448x over previous
import functools

import jax
import jax.numpy as jnp
from jax import lax
from jax.experimental import pallas as pl
from jax.experimental.pallas import tpu as pltpu
from jax.experimental.pallas import tpu_sc as plsc

_H = 16
_S = 2048
_NB = 15
_RB = _S // 8
_W = 4088
_VAR0 = 1984
_FILL_R = 2096

_mesh = plsc.VectorSubcoreMesh(
    core_axis_name="c", subcore_axis_name="s", num_cores=2, num_subcores=16
)


@functools.partial(
    pl.kernel,
    out_type=jax.ShapeDtypeStruct((_H, _S, _S), jnp.float32),
    mesh=_mesh,
    scratch_types=[
        pltpu.VMEM((_NB * _NB * _H,), jnp.float32),
        pltpu.VMEM((8, _W), jnp.float32),
        pltpu.SemaphoreType.DMA,
    ],
    compiler_params=pltpu.CompilerParams(
        needs_layout_passes=False, use_tc_tiling_on_sc=False
    ),
)
def _rpb_sc(table_hbm, out_hbm, table_v, strip_v, sem):
    h = lax.axis_index("s")
    half = lax.axis_index("c")

    pltpu.sync_copy(table_hbm, table_v)

    iota = lax.iota(jnp.int32, 16)
    fj = iota & 7
    ksub = iota >> 3

    def tab_idx(fi, fjv, idx1):
        return (fi - fjv + 7) * (_NB * _H) + idx1 * _H + h

    pat_l = [plsc.load_gather(table_v, [tab_idx(fi, fj, 14)]) for fi in range(8)]
    pat_r = [plsc.load_gather(table_v, [tab_idx(fi, fj, 0)]) for fi in range(8)]

    for fi in range(8):
        for u in range(7):
            idx1 = 14 - (2 * u + ksub)
            vals = plsc.load_gather(table_v, [tab_idx(fi, fj, idx1)])
            strip_v[fi, pl.ds(_VAR0 + 16 * u, 16)] = vals

    def fill_body(t, carry):
        c = 16 * t
        for fi in range(8):
            strip_v[fi, pl.ds(c, 16)] = pat_l[fi]
            strip_v[fi, pl.ds(_FILL_R + c, 16)] = pat_r[fi]
        return carry

    lax.fori_loop(0, _VAR0 // 16, fill_body, 0)
    for fi in range(8):
        strip_v[fi, pl.ds(4072, 16)] = pat_r[fi]

    r0 = half * (_RB // 2)

    def window_copy(ri):
        c0 = 8 * (_RB - 1 - ri)
        return pltpu.make_async_copy(
            strip_v.at[:, pl.ds(c0, _S)],
            out_hbm.at[h, pl.ds(8 * ri, 8), :],
            sem,
        )

    def write_body(t, carry):
        window_copy(r0 + t).start()

        @pl.when(t >= 3)
        def _():
            window_copy(r0).wait()

        return carry

    lax.fori_loop(0, _RB // 2, write_body, 0)
    for _ in range(3):
        window_copy(r0).wait()


def kernel(seq_len, table):
    del seq_len
    out = _rpb_sc(table.reshape(-1))
    return out[None]
